# baseline (speedup 1.0000x reference)
import jax
import jax.numpy as jnp
from jax.experimental import pallas as pl


def _gat_conv(x, edge_index, W, a_src, a_dst, b):
    N = x.shape[0]
    H, C = a_src.shape
    xp = (x @ W).reshape(N, H, C)
    loops = jnp.arange(N, dtype=edge_index.dtype)
    src = jnp.concatenate([edge_index[0], loops])
    dst = jnp.concatenate([edge_index[1], loops])
    alpha_src = (xp * a_src[None, :, :]).sum(-1)
    alpha_dst = (xp * a_dst[None, :, :]).sum(-1)
    e = alpha_src[src] + alpha_dst[dst]
    e = jax.nn.leaky_relu(e, 0.2)
    m = jax.ops.segment_max(e, dst, num_segments=N)
    m = jnp.where(jnp.isfinite(m), m, 0.0)
    ex = jnp.exp(e - m[dst])
    den = jax.ops.segment_sum(ex, dst, num_segments=N)
    alpha = ex / (den[dst] + 1e-16)
    out = jax.ops.segment_sum(xp[src] * alpha[:, :, None], dst, num_segments=N)
    return out.mean(axis=1) + b[None, :]


def _identity_kernel(x_ref, o_ref):
    o_ref[...] = x_ref[...]


def kernel(x, adj, W1, a_src1, a_dst1, b1, W2, a_src2, a_dst2, b2, W3, a_src3, a_dst3, b3):
    x = pl.pallas_call(
        _identity_kernel,
        out_shape=jax.ShapeDtypeStruct(x.shape, x.dtype),
    )(x)
    h1 = jax.nn.relu(_gat_conv(x, adj, W1, a_src1, a_dst1, b1))
    mu = _gat_conv(h1, adj, W2, a_src2, a_dst2, b2)
    logvar = _gat_conv(h1, adj, W3, a_src3, a_dst3, b3)
    z = mu
    return (z, mu, logvar)


# speedup vs baseline: 21.4416x; 21.4416x over previous
import functools

import jax
import jax.numpy as jnp
from jax import lax
from jax.experimental import pallas as pl
from jax.experimental.pallas import tpu as pltpu
from jax.experimental.pallas import tpu_sc as plsc

N = 10000
E = 320000
D_IN = 128
H1 = 64
H2 = 32
HEADS = 5

N1 = 10240
BN = 512
NBLK = N1 // BN
K = 128
NW = 32
E1 = E + N
EPAD = ((E1 + NW * K - 1) // (NW * K)) * (NW * K)
EPW = EPAD // NW
NCH = EPW // K
RPT = N1 // 16
ZR = 160
EPS = 1e-16


def _tc_prep_body(nh, C, x_ref, W_ref, av_ref, bv_ref, *out_refs):
    _prep_from_block(x_ref[...], W_ref, av_ref, bv_ref, out_refs)


def _tc_prep(x, W, a_src, a_dst, C):
    din = x.shape[1]
    nh = HEADS
    out_shapes = ([jax.ShapeDtypeStruct((N1, C), jnp.float32)] * nh
                  + [jax.ShapeDtypeStruct((N1, 16), jnp.float32)] * 2)
    out_specs = ([pl.BlockSpec((BN, C), lambda i: (i, 0))] * nh
                 + [pl.BlockSpec((BN, 16), lambda i: (i, 0))] * 2)
    outs = pl.pallas_call(
        functools.partial(_tc_prep_body, nh, C),
        grid=(NBLK,),
        in_specs=[
            pl.BlockSpec((BN, din), lambda i: (i, 0)),
            pl.BlockSpec((din, nh * C), lambda i: (0, 0)),
            pl.BlockSpec((nh, C), lambda i: (0, 0)),
            pl.BlockSpec((nh, C), lambda i: (0, 0)),
        ],
        out_specs=out_specs,
        out_shape=out_shapes,
    )(x, W, a_src, a_dst)
    return list(outs[:nh]), outs[nh], outs[nh + 1]


def _combine(outp, den, brow):
    dsum = den[0] + den[1]
    acc = None
    for h in range(HEADS):
        oh = outp[0, h] + outp[1, h]
        dh = dsum[:, h:h + 1] + EPS
        term = oh / dh
        acc = term if acc is None else acc + term
    return acc * (1.0 / HEADS) + brow


def kernel(x, adj, W1, a_src1, a_dst1, b1, W2, a_src2, a_dst2, b2,
           W3, a_src3, a_dst3, b3):
    adj32 = adj.astype(jnp.int32)
    loops = jnp.arange(N, dtype=jnp.int32)
    padv = jnp.full((EPAD - E1,), N, jnp.int32)
    src = jnp.concatenate([adj32[0], loops, padv])
    dst = jnp.concatenate([adj32[1], loops, padv])
    xpad = jnp.pad(x, ((0, N1 - N), (0, 0)))

    xh1, as1, ad1 = _tc_prep(xpad, W1, a_src1, a_dst1, H1)
    outp1, den1 = _sc_edge(src, dst, as1, ad1, xh1, H1)

    mid_out_shapes = ([jax.ShapeDtypeStruct((N1, H2), jnp.float32)] * HEADS
                      + [jax.ShapeDtypeStruct((N1, 16), jnp.float32)] * 2) * 2
    mid_out_specs = ([pl.BlockSpec((BN, H2), lambda i: (i, 0))] * HEADS
                     + [pl.BlockSpec((BN, 16), lambda i: (i, 0))] * 2) * 2
    mid = pl.pallas_call(
        _tc_mid_wrap,
        grid=(NBLK,),
        in_specs=[
            pl.BlockSpec((2, HEADS, BN, H1), lambda i: (0, 0, i, 0)),
            pl.BlockSpec((2, BN, 16), lambda i: (0, i, 0)),
            pl.BlockSpec((1, H1), lambda i: (0, 0)),
            pl.BlockSpec((H1, HEADS * H2), lambda i: (0, 0)),
            pl.BlockSpec((HEADS, H2), lambda i: (0, 0)),
            pl.BlockSpec((HEADS, H2), lambda i: (0, 0)),
            pl.BlockSpec((H1, HEADS * H2), lambda i: (0, 0)),
            pl.BlockSpec((HEADS, H2), lambda i: (0, 0)),
            pl.BlockSpec((HEADS, H2), lambda i: (0, 0)),
        ],
        out_specs=mid_out_specs,
        out_shape=mid_out_shapes,
    )(outp1, den1, b1[None, :], W2, a_src2, a_dst2, W3, a_src3, a_dst3)
    xh2, as2, ad2 = list(mid[:HEADS]), mid[HEADS], mid[HEADS + 1]
    xh3, as3, ad3 = list(mid[HEADS + 2:2 * HEADS + 2]), mid[-2], mid[-1]

    outp2, den2 = _sc_edge(src, dst, as2, ad2, xh2, H2)
    outp3, den3 = _sc_edge(src, dst, as3, ad3, xh3, H2)

    mu_pad, lv_pad = pl.pallas_call(
        _tc_final_body,
        grid=(NBLK,),
        in_specs=[
            pl.BlockSpec((2, HEADS, BN, H2), lambda i: (0, 0, i, 0)),
            pl.BlockSpec((2, BN, 16), lambda i: (0, i, 0)),
            pl.BlockSpec((1, H2), lambda i: (0, 0)),
            pl.BlockSpec((2, HEADS, BN, H2), lambda i: (0, 0, i, 0)),
            pl.BlockSpec((2, BN, 16), lambda i: (0, i, 0)),
            pl.BlockSpec((1, H2), lambda i: (0, 0)),
        ],
        out_specs=[pl.BlockSpec((BN, H2), lambda i: (i, 0))] * 2,
        out_shape=[jax.ShapeDtypeStruct((N1, H2), jnp.float32)] * 2,
    )(outp2, den2, b2[None, :], outp3, den3, b3[None, :])
    mu = mu_pad[:N]
    logvar = lv_pad[:N]
    return (mu, mu, logvar)


def _tc_mid_wrap(outp_ref, den_ref, b1_ref, W2_ref, a2s_ref, a2d_ref,
                 W3_ref, a3s_ref, a3d_ref, *out_refs):
    h1 = jnp.maximum(_combine(outp_ref[...], den_ref[...], b1_ref[...]), 0.0)
    _prep_from_block(h1, W2_ref, a2s_ref, a2d_ref, out_refs[:HEADS + 2])
    _prep_from_block(h1, W3_ref, a3s_ref, a3d_ref, out_refs[HEADS + 2:])


def _prep_from_block(xb, W_ref, av_ref, bv_ref, out_refs):
    nh = HEADS
    C = av_ref.shape[1]
    xh_refs = out_refs[:nh]
    as_ref, ad_ref = out_refs[nh], out_refs[nh + 1]
    dn = (((1,), (1,)), ((), ()))
    acols, bcols = [], []
    for h in range(nh):
        xh = jnp.dot(xb, W_ref[:, h * C:(h + 1) * C],
                     preferred_element_type=jnp.float32)
        xh_refs[h][...] = xh
        acols.append(lax.dot_general(xh, av_ref[h:h + 1, :], dn,
                                     preferred_element_type=jnp.float32))
        bcols.append(lax.dot_general(xh, bv_ref[h:h + 1, :], dn,
                                     preferred_element_type=jnp.float32))
    z = jnp.zeros((xb.shape[0], 16 - nh), jnp.float32)
    as_ref[...] = jnp.concatenate(acols + [z], axis=1)
    ad_ref[...] = jnp.concatenate(bcols + [z], axis=1)


def _tc_final_body(outp2_ref, den2_ref, b2_ref, outp3_ref, den3_ref, b3_ref,
                   mu_ref, lv_ref):
    mu_ref[...] = _combine(outp2_ref[...], den2_ref[...], b2_ref[...])
    lv_ref[...] = _combine(outp3_ref[...], den3_ref[...], b3_ref[...])


def _sc_edge(src, dst, as_tab, ad_tab, xh_list, C):
    mesh = plsc.VectorSubcoreMesh(core_axis_name="c", subcore_axis_name="s")

    @functools.partial(
        pl.kernel,
        mesh=mesh,
        compiler_params=pltpu.CompilerParams(use_tc_tiling_on_sc=False),
        out_type=[
            jax.ShapeDtypeStruct((2, HEADS, N1, C), jnp.float32),
            jax.ShapeDtypeStruct((2, N1, 16), jnp.float32),
            jax.ShapeDtypeStruct((EPAD, 16), jnp.float32),
        ],
        scratch_types=[
            pltpu.VMEM_SHARED((N1, C), jnp.float32),
            pltpu.VMEM_SHARED((N1, 16), jnp.float32),
            pltpu.VMEM((K,), jnp.int32),
            pltpu.VMEM((K,), jnp.int32),
            pltpu.VMEM((K, 16), jnp.float32),
            pltpu.VMEM((K, 16), jnp.float32),
            pltpu.VMEM((K, 16), jnp.float32),
            pltpu.VMEM((K, C), jnp.float32),
            pltpu.VMEM((ZR, C), jnp.float32),
            pltpu.VMEM((ZR, 16), jnp.float32),
            pltpu.SemaphoreType.DMA,
            pltpu.SemaphoreType.DMA,
        ],
    )
    def k(src_hbm, dst_hbm, as_hbm, ad_hbm, x0, x1, x2, x3, x4,
          outp_hbm, den_hbm, w_hbm,
          out_sp, den_sp, srcv, dstv, asr, adr, wvr, rowsv, zbo, zbd,
          sem, sem2):
        xh_hbm = (x0, x1, x2, x3, x4)
        cid = lax.axis_index("c")
        sid = lax.axis_index("s")
        wid = cid * 16 + sid
        base = wid * EPW
        lo = sid * RPT

        zvec16 = jnp.zeros((16,), jnp.float32)

        def zfill(j, _):
            for cc in range(C // 16):
                zbo[j, pl.ds(cc * 16, 16)] = zvec16
            zbd[j, :] = zvec16
            return 0
        lax.fori_loop(0, ZR, zfill, 0, unroll=4)

        for t in range(RPT // ZR):
            pltpu.sync_copy(zbd, den_sp.at[pl.ds(lo + t * ZR, ZR)])
        plsc.subcore_barrier()

        def chunk_b(i, _):
            off = base + i * K
            pltpu.sync_copy(src_hbm.at[pl.ds(off, K)], srcv)
            pltpu.sync_copy(dst_hbm.at[pl.ds(off, K)], dstv)
            pltpu.async_copy(as_hbm.at[srcv], asr, sem).wait()
            pltpu.async_copy(ad_hbm.at[dstv], adr, sem2).wait()

            def ej(j, _):
                e = asr[j, :] + adr[j, :]
                e = jnp.where(e >= 0.0, e, 0.2 * e)
                wvr[j, :] = jnp.exp(e)
                return 0
            lax.fori_loop(0, K, ej, 0, unroll=4)
            pltpu.sync_copy(wvr, w_hbm.at[pl.ds(off, K)])
            pltpu.sync_copy(wvr, den_sp.at[dstv], add=True)
            return 0
        lax.fori_loop(0, NCH, chunk_b, 0)
        plsc.subcore_barrier()
        pltpu.sync_copy(den_sp.at[pl.ds(lo, RPT)],
                        den_hbm.at[cid, pl.ds(lo, RPT)])

        for h in range(HEADS):
            for t in range(RPT // ZR):
                pltpu.sync_copy(zbo, out_sp.at[pl.ds(lo + t * ZR, ZR)])
            plsc.subcore_barrier()

            def chunk_c(i, _):
                off = base + i * K
                pltpu.sync_copy(src_hbm.at[pl.ds(off, K)], srcv)
                pltpu.sync_copy(dst_hbm.at[pl.ds(off, K)], dstv)
                pltpu.sync_copy(w_hbm.at[pl.ds(off, K)], wvr)
                pltpu.async_copy(xh_hbm[h].at[srcv], rowsv, sem).wait()

                def ej(j, _):
                    wb = wvr[j, :][h]
                    for cc in range(C // 16):
                        sl = pl.ds(cc * 16, 16)
                        rowsv[j, sl] = rowsv[j, sl] * wb
                    return 0
                lax.fori_loop(0, K, ej, 0, unroll=2)
                pltpu.sync_copy(rowsv, out_sp.at[dstv], add=True)
                return 0
            lax.fori_loop(0, NCH, chunk_c, 0)
            plsc.subcore_barrier()
            pltpu.sync_copy(out_sp.at[pl.ds(lo, RPT)],
                            outp_hbm.at[cid, h, pl.ds(lo, RPT)])

    outp, den, _w = k(src, dst, as_tab, ad_tab, *xh_list)
    return outp, den


# speedup vs baseline: 55.2577x; 2.5771x over previous
import functools

import jax
import jax.numpy as jnp
from jax import lax
from jax.experimental import pallas as pl
from jax.experimental.pallas import tpu as pltpu
from jax.experimental.pallas import tpu_sc as plsc

N = 10000
E = 320000
D_IN = 128
H1 = 64
H2 = 32
HEADS = 5

N1 = 10240
BN = 512
NBLK = N1 // BN
K = 128
NW = 32
E1 = E + N
EPAD = ((E1 + NW * K - 1) // (NW * K)) * (NW * K)
EPW = EPAD // NW
NCH = EPW // K
RPT = N1 // 16
ZR = 160
EPS = 1e-16


def _tc_prep_body(nh, C, x_ref, W_ref, av_ref, bv_ref, *out_refs):
    _prep_from_block(x_ref[...], W_ref, av_ref, bv_ref, out_refs)


def _tc_prep(x, W, a_src, a_dst, C):
    din = x.shape[1]
    nh = HEADS
    out_shapes = ([jax.ShapeDtypeStruct((N1, C), jnp.float32)] * nh
                  + [jax.ShapeDtypeStruct((N1, 16), jnp.float32)] * 2)
    out_specs = ([pl.BlockSpec((BN, C), lambda i: (i, 0))] * nh
                 + [pl.BlockSpec((BN, 16), lambda i: (i, 0))] * 2)
    outs = pl.pallas_call(
        functools.partial(_tc_prep_body, nh, C),
        grid=(NBLK,),
        in_specs=[
            pl.BlockSpec((BN, din), lambda i: (i, 0)),
            pl.BlockSpec((din, nh * C), lambda i: (0, 0)),
            pl.BlockSpec((nh, C), lambda i: (0, 0)),
            pl.BlockSpec((nh, C), lambda i: (0, 0)),
        ],
        out_specs=out_specs,
        out_shape=out_shapes,
    )(x, W, a_src, a_dst)
    return list(outs[:nh]), outs[nh], outs[nh + 1]


def _combine(outp, den, brow):
    dsum = den[0] + den[1]
    acc = None
    for h in range(HEADS):
        oh = outp[0, h] + outp[1, h]
        dh = dsum[:, h:h + 1] + EPS
        term = oh / dh
        acc = term if acc is None else acc + term
    return acc * (1.0 / HEADS) + brow


def kernel(x, adj, W1, a_src1, a_dst1, b1, W2, a_src2, a_dst2, b2,
           W3, a_src3, a_dst3, b3):
    adj32 = adj.astype(jnp.int32)
    loops = jnp.arange(N, dtype=jnp.int32)
    padv = jnp.full((EPAD - E1,), N, jnp.int32)
    src = jnp.concatenate([adj32[0], loops, padv])
    dst = jnp.concatenate([adj32[1], loops, padv])
    xpad = jnp.pad(x, ((0, N1 - N), (0, 0)))

    xh1, as1, ad1 = _tc_prep(xpad, W1, a_src1, a_dst1, H1)
    outp1, den1 = _sc_edge(src, dst, as1, ad1, xh1, H1)

    mid_out_shapes = ([jax.ShapeDtypeStruct((N1, H2), jnp.float32)] * HEADS
                      + [jax.ShapeDtypeStruct((N1, 16), jnp.float32)] * 2) * 2
    mid_out_specs = ([pl.BlockSpec((BN, H2), lambda i: (i, 0))] * HEADS
                     + [pl.BlockSpec((BN, 16), lambda i: (i, 0))] * 2) * 2
    mid = pl.pallas_call(
        _tc_mid_wrap,
        grid=(NBLK,),
        in_specs=[
            pl.BlockSpec((2, HEADS, BN, H1), lambda i: (0, 0, i, 0)),
            pl.BlockSpec((2, BN, 16), lambda i: (0, i, 0)),
            pl.BlockSpec((1, H1), lambda i: (0, 0)),
            pl.BlockSpec((H1, HEADS * H2), lambda i: (0, 0)),
            pl.BlockSpec((HEADS, H2), lambda i: (0, 0)),
            pl.BlockSpec((HEADS, H2), lambda i: (0, 0)),
            pl.BlockSpec((H1, HEADS * H2), lambda i: (0, 0)),
            pl.BlockSpec((HEADS, H2), lambda i: (0, 0)),
            pl.BlockSpec((HEADS, H2), lambda i: (0, 0)),
        ],
        out_specs=mid_out_specs,
        out_shape=mid_out_shapes,
    )(outp1, den1, b1[None, :], W2, a_src2, a_dst2, W3, a_src3, a_dst3)
    xh2, as2, ad2 = list(mid[:HEADS]), mid[HEADS], mid[HEADS + 1]
    xh3, as3, ad3 = list(mid[HEADS + 2:2 * HEADS + 2]), mid[-2], mid[-1]

    outp2, den2 = _sc_edge(src, dst, as2, ad2, xh2, H2)
    outp3, den3 = _sc_edge(src, dst, as3, ad3, xh3, H2)

    mu_pad, lv_pad = pl.pallas_call(
        _tc_final_body,
        grid=(NBLK,),
        in_specs=[
            pl.BlockSpec((2, HEADS, BN, H2), lambda i: (0, 0, i, 0)),
            pl.BlockSpec((2, BN, 16), lambda i: (0, i, 0)),
            pl.BlockSpec((1, H2), lambda i: (0, 0)),
            pl.BlockSpec((2, HEADS, BN, H2), lambda i: (0, 0, i, 0)),
            pl.BlockSpec((2, BN, 16), lambda i: (0, i, 0)),
            pl.BlockSpec((1, H2), lambda i: (0, 0)),
        ],
        out_specs=[pl.BlockSpec((BN, H2), lambda i: (i, 0))] * 2,
        out_shape=[jax.ShapeDtypeStruct((N1, H2), jnp.float32)] * 2,
    )(outp2, den2, b2[None, :], outp3, den3, b3[None, :])
    mu = mu_pad[:N]
    logvar = lv_pad[:N]
    return (mu, mu, logvar)


def _tc_mid_wrap(outp_ref, den_ref, b1_ref, W2_ref, a2s_ref, a2d_ref,
                 W3_ref, a3s_ref, a3d_ref, *out_refs):
    h1 = jnp.maximum(_combine(outp_ref[...], den_ref[...], b1_ref[...]), 0.0)
    _prep_from_block(h1, W2_ref, a2s_ref, a2d_ref, out_refs[:HEADS + 2])
    _prep_from_block(h1, W3_ref, a3s_ref, a3d_ref, out_refs[HEADS + 2:])


def _prep_from_block(xb, W_ref, av_ref, bv_ref, out_refs):
    nh = HEADS
    C = av_ref.shape[1]
    xh_refs = out_refs[:nh]
    as_ref, ad_ref = out_refs[nh], out_refs[nh + 1]
    dn = (((1,), (1,)), ((), ()))
    acols, bcols = [], []
    for h in range(nh):
        xh = jnp.dot(xb, W_ref[:, h * C:(h + 1) * C],
                     preferred_element_type=jnp.float32)
        xh_refs[h][...] = xh
        acols.append(lax.dot_general(xh, av_ref[h:h + 1, :], dn,
                                     preferred_element_type=jnp.float32))
        bcols.append(lax.dot_general(xh, bv_ref[h:h + 1, :], dn,
                                     preferred_element_type=jnp.float32))
    z = jnp.zeros((xb.shape[0], 16 - nh), jnp.float32)
    as_ref[...] = jnp.concatenate(acols + [z], axis=1)
    ad_ref[...] = jnp.concatenate(bcols + [z], axis=1)


def _tc_final_body(outp2_ref, den2_ref, b2_ref, outp3_ref, den3_ref, b3_ref,
                   mu_ref, lv_ref):
    mu_ref[...] = _combine(outp2_ref[...], den2_ref[...], b2_ref[...])
    lv_ref[...] = _combine(outp3_ref[...], den3_ref[...], b3_ref[...])


def _sc_edge(src, dst, as_tab, ad_tab, xh_list, C):
    mesh = plsc.VectorSubcoreMesh(core_axis_name="c", subcore_axis_name="s")
    NT = NCH // 3

    @functools.partial(
        pl.kernel,
        mesh=mesh,
        compiler_params=pltpu.CompilerParams(use_tc_tiling_on_sc=False),
        out_type=[
            jax.ShapeDtypeStruct((2, HEADS, N1, C), jnp.float32),
            jax.ShapeDtypeStruct((2, N1, 16), jnp.float32),
            jax.ShapeDtypeStruct((EPAD // K, K, 16), jnp.float32),
        ],
        scratch_types=[
            pltpu.VMEM_SHARED((N1, C), jnp.float32),
            pltpu.VMEM_SHARED((N1, 16), jnp.float32),
            pltpu.VMEM((NCH, K), jnp.int32),
            pltpu.VMEM((NCH, K), jnp.int32),
            pltpu.VMEM((3, K, 16), jnp.float32),
            pltpu.VMEM((3, K, 16), jnp.float32),
            pltpu.VMEM((3, K, 16), jnp.float32),
            pltpu.VMEM((3, K, C), jnp.float32),
            pltpu.VMEM((ZR, C), jnp.float32),
            pltpu.VMEM((ZR, 16), jnp.float32),
            pltpu.VMEM((K,), jnp.int32),
            pltpu.VMEM((K,), jnp.int32),
            pltpu.VMEM((K,), jnp.int32),
        ] + [pltpu.SemaphoreType.DMA] * 12,
    )
    def k(src_hbm, dst_hbm, as_hbm, ad_hbm, x0, x1, x2, x3, x4,
          outp_hbm, den_hbm, w_hbm,
          out_sp, den_sp, srcv, dstv, asr, adr, wvr, rowsv, zbo, zbd,
          dst0, dst1, dst2,
          sa0, sa1, sa2, sb0, sb1, sb2, sc0, sc1, sc2, sd0, sd1, sd2):
        xh_hbm = (x0, x1, x2, x3, x4)
        dst_st = (dst0, dst1, dst2)
        sem_a = (sa0, sa1, sa2)
        sem_b = (sb0, sb1, sb2)
        sem_c = (sc0, sc1, sc2)
        sem_d = (sd0, sd1, sd2)
        cid = lax.axis_index("c")
        sid = lax.axis_index("s")
        wid = cid * 16 + sid
        cbase = wid * NCH
        lo = sid * RPT

        pltpu.sync_copy(src_hbm.at[pl.ds(cbase, NCH)], srcv)
        pltpu.sync_copy(dst_hbm.at[pl.ds(cbase, NCH)], dstv)

        zvec16 = jnp.zeros((16,), jnp.float32)

        def zfill(j, _):
            for cc in range(C // 16):
                zbo[j, pl.ds(cc * 16, 16)] = zvec16
            zbd[j, :] = zvec16
            return 0
        lax.fori_loop(0, ZR, zfill, 0, unroll=4)

        for t in range(RPT // ZR):
            pltpu.sync_copy(zbd, den_sp.at[pl.ds(lo + t * ZR, ZR)])
        plsc.subcore_barrier()

        def b_gather(i, b):
            pltpu.async_copy(as_hbm.at[srcv.at[i]], asr.at[b], sem_a[b])
            pltpu.async_copy(ad_hbm.at[dstv.at[i]], adr.at[b], sem_b[b])
            pltpu.async_copy(dst_hbm.at[cbase + i], dst_st[b], sem_d[b])

        b_gather(0, 0)
        b_gather(1, 1)

        def b_slot(t, b):
            i = 3 * t + b
            pltpu.make_async_copy(as_hbm.at[srcv.at[i]], asr.at[b],
                                  sem_a[b]).wait()
            pltpu.make_async_copy(ad_hbm.at[dstv.at[i]], adr.at[b],
                                  sem_b[b]).wait()

            def ej(j, _):
                e = asr[b, j, :] + adr[b, j, :]
                e = jnp.where(e >= 0.0, e, 0.2 * e)
                wvr[b, j, :] = jnp.exp(e)
                return 0
            lax.fori_loop(0, K, ej, 0, unroll=8)
            pltpu.async_copy(wvr.at[b], w_hbm.at[cbase + i], sem_c[b])
            pltpu.make_async_copy(dst_hbm.at[cbase + i], dst_st[b],
                                  sem_d[b]).wait()
            pltpu.sync_copy(wvr.at[b], den_sp.at[dst_st[b]], add=True)
            b2 = (b + 2) % 3
            if b == 0:
                @pl.when(t > 0)
                def _():
                    pltpu.make_async_copy(wvr.at[b2], w_hbm.at[cbase + i - 1],
                                          sem_c[b2]).wait()

                b_gather(i + 2, b2)
            elif b == 2:
                pltpu.make_async_copy(wvr.at[b2], w_hbm.at[cbase + i - 1],
                                      sem_c[b2]).wait()

                @pl.when(t < NT - 1)
                def _():
                    b_gather(i + 2, b2)
            else:
                pltpu.make_async_copy(wvr.at[b2], w_hbm.at[cbase + i - 1],
                                      sem_c[b2]).wait()

                @pl.when(t < NT - 1)
                def _():
                    b_gather(i + 2, b2)

        def b_body(t, _):
            for b in range(3):
                b_slot(t, b)
            return 0
        lax.fori_loop(0, NT, b_body, 0)
        pltpu.make_async_copy(wvr.at[2], w_hbm.at[cbase + NCH - 1],
                              sem_c[2]).wait()
        plsc.subcore_barrier()
        pltpu.sync_copy(den_sp.at[pl.ds(lo, RPT)],
                        den_hbm.at[cid, pl.ds(lo, RPT)])

        for h in range(HEADS):
            for t in range(RPT // ZR):
                pltpu.sync_copy(zbo, out_sp.at[pl.ds(lo + t * ZR, ZR)])
            plsc.subcore_barrier()

            def c_gather(i, b):
                pltpu.async_copy(xh_hbm[h].at[srcv.at[i]], rowsv.at[b],
                                 sem_a[b])
                pltpu.async_copy(w_hbm.at[cbase + i], wvr.at[b], sem_b[b])
                pltpu.async_copy(dst_hbm.at[cbase + i], dst_st[b], sem_d[b])

            c_gather(0, 0)
            c_gather(1, 1)

            def c_slot(t, b):
                i = 3 * t + b
                pltpu.make_async_copy(xh_hbm[h].at[srcv.at[i]], rowsv.at[b],
                                      sem_a[b]).wait()
                pltpu.make_async_copy(w_hbm.at[cbase + i], wvr.at[b],
                                      sem_b[b]).wait()

                def ej(j, _):
                    wb = wvr[b, j, :][h]
                    for cc in range(C // 16):
                        sl = pl.ds(cc * 16, 16)
                        rowsv[b, j, sl] = rowsv[b, j, sl] * wb
                    return 0
                lax.fori_loop(0, K, ej, 0, unroll=8)
                pltpu.make_async_copy(dst_hbm.at[cbase + i], dst_st[b],
                                      sem_d[b]).wait()
                pltpu.async_copy(rowsv.at[b], out_sp.at[dst_st[b]],
                                 sem_c[b], add=True)
                b2 = (b + 2) % 3

                def drain(prev_i):
                    pltpu.make_async_copy(rowsv.at[b2],
                                          out_sp.at[dst_st[b2]],
                                          sem_c[b2]).wait()

                if b == 0:
                    @pl.when(t > 0)
                    def _():
                        drain(i - 1)
                    c_gather(i + 2, b2)
                elif b == 2:
                    drain(i - 1)

                    @pl.when(t < NT - 1)
                    def _():
                        c_gather(i + 2, b2)
                else:
                    drain(i - 1)

                    @pl.when(t < NT - 1)
                    def _():
                        c_gather(i + 2, b2)

            def c_body(t, _):
                for b in range(3):
                    c_slot(t, b)
                return 0
            lax.fori_loop(0, NT, c_body, 0)
            pltpu.make_async_copy(rowsv.at[2], out_sp.at[dst_st[2]],
                                  sem_c[2]).wait()
            plsc.subcore_barrier()
            pltpu.sync_copy(out_sp.at[pl.ds(lo, RPT)],
                            outp_hbm.at[cid, h, pl.ds(lo, RPT)])

    outp, den, _w = k(src.reshape(EPAD // K, K), dst.reshape(EPAD // K, K),
                      as_tab, ad_tab, *xh_list)
    return outp, den


# speedup vs baseline: 66.4270x; 1.2021x over previous
import functools

import jax
import jax.numpy as jnp
from jax import lax
from jax.experimental import pallas as pl
from jax.experimental.pallas import tpu as pltpu
from jax.experimental.pallas import tpu_sc as plsc

N = 10000
E = 320000
D_IN = 128
H1 = 64
H2 = 32
HEADS = 5

N1 = 10240
BN = 512
NBLK = N1 // BN
K = 128
NW = 32
E1 = E + N
EPAD = ((E1 + NW * K - 1) // (NW * K)) * (NW * K)
NCHT = EPAD // K
NCH = NCHT // NW
NT = NCH // 3
RPT = N1 // 16
ZR = 160
EPS = 1e-16

L1_LANES = ((0,), (1,), (2,), (3,), (4,))
L23_LANES = tuple((h, 8 + h) for h in range(HEADS))


def _prep_block(xb, Wg_ref, apad_ref, bpad_ref, score_map, C, out_refs):
    ngroups = len(out_refs) - 2
    as_ref, ad_ref = out_refs[ngroups], out_refs[ngroups + 1]
    xg = []
    off = 0
    for g in range(ngroups):
        wg = out_refs[g].shape[-1]
        x_out = jnp.dot(xb, Wg_ref[:, off:off + wg],
                        preferred_element_type=jnp.float32)
        out_refs[g][...] = x_out
        xg.append(x_out)
        off += wg
    dn = (((1,), (1,)), ((), ()))
    zcol = jnp.zeros((xb.shape[0], 1), jnp.float32)
    acols, bcols = [zcol] * 16, [zcol] * 16
    for lane, g, coff in score_map:
        blk = xg[g][:, coff:coff + C]
        acols[lane] = lax.dot_general(blk, apad_ref[lane:lane + 1, :], dn,
                                      preferred_element_type=jnp.float32)
        bcols[lane] = lax.dot_general(blk, bpad_ref[lane:lane + 1, :], dn,
                                      preferred_element_type=jnp.float32)
    as_ref[...] = jnp.concatenate(acols, axis=1)
    ad_ref[...] = jnp.concatenate(bcols, axis=1)


def _combine(groups, lanes, cw, den, eps_lanes):
    dsum = den[0] + den[1]
    acc = None
    for (g, bi, lane) in eps_lanes:
        oh = groups[g][0][:, bi * cw:(bi + 1) * cw] \
            + groups[g][1][:, bi * cw:(bi + 1) * cw]
        dh = dsum[:, lane:lane + 1] + EPS
        term = oh / dh
        acc = term if acc is None else acc + term
    return acc * (1.0 / HEADS)


def _tc_mid_body(og0, og1, og2, og3, og4, den_ref, b1_ref, W23_ref,
                 apad_ref, bpad_ref, *out_refs):
    groups = [(og[0], og[1]) for og in (og0, og1, og2, og3, og4)]
    triples = [(h, 0, h) for h in range(HEADS)]
    h1 = _combine(groups, L1_LANES, H1, den_ref[...], triples) + b1_ref[...]
    h1 = jnp.maximum(h1, 0.0)
    score_map = tuple([(h, h, 0) for h in range(HEADS)]
                      + [(8 + h, h, H2) for h in range(HEADS)])
    _prep_block(h1, W23_ref, apad_ref, bpad_ref, score_map, H2, out_refs)


def _tc_final_body(g0, g1, g2, g3, g4, den_ref, b2_ref, b3_ref,
                   mu_ref, lv_ref):
    groups = [(g0[0], g0[1]), (g1[0], g1[1]), (g2[0], g2[1]),
              (g3[0], g3[1]), (g4[0], g4[1])]
    den = den_ref[...]
    mu_t = [(h, 0, h) for h in range(HEADS)]
    lv_t = [(h, 1, 8 + h) for h in range(HEADS)]
    mu_ref[...] = _combine(groups, L23_LANES, H2, den, mu_t) + b2_ref[...]
    lv_ref[...] = _combine(groups, L23_LANES, H2, den, lv_t) + b3_ref[...]


def kernel(x, adj, W1, a_src1, a_dst1, b1, W2, a_src2, a_dst2, b2,
           W3, a_src3, a_dst3, b3):
    f32 = jnp.float32
    adj32 = adj.astype(jnp.int32)
    loops = jnp.arange(N, dtype=jnp.int32)
    padv = jnp.full((EPAD - E1,), N, jnp.int32)
    src = jnp.concatenate([adj32[0], loops, padv]).reshape(NCHT, K)
    dst = jnp.concatenate([adj32[1], loops, padv]).reshape(NCHT, K)
    xpad = jnp.pad(x, ((0, N1 - N), (0, 0)))
    Wg1 = W1
    a1p = jnp.zeros((16, H1), f32).at[:HEADS].set(a_src1)
    b1p = jnp.zeros((16, H1), f32).at[:HEADS].set(a_dst1)
    W23 = jnp.concatenate([W2.reshape(H1, HEADS, H2),
                           W3.reshape(H1, HEADS, H2)], axis=2)
    W23 = W23.reshape(H1, 2 * HEADS * H2)
    a23p = jnp.zeros((16, H2), f32).at[:HEADS].set(a_src2).at[8:8 + HEADS].set(a_src3)
    b23p = jnp.zeros((16, H2), f32).at[:HEADS].set(a_dst2).at[8:8 + HEADS].set(a_dst3)

    score_map1 = tuple((h, h, 0) for h in range(HEADS))
    gw1 = (H1,) * HEADS
    prep_out = pl.pallas_call(
        functools.partial(_tc_prep_wrap, score_map1, H1, gw1),
        grid=(NBLK,),
        in_specs=[
            pl.BlockSpec((BN, D_IN), lambda i: (i, 0)),
            pl.BlockSpec((D_IN, sum(gw1)), lambda i: (0, 0)),
            pl.BlockSpec((16, H1), lambda i: (0, 0)),
            pl.BlockSpec((16, H1), lambda i: (0, 0)),
        ],
        out_specs=[pl.BlockSpec((BN, w), lambda i: (i, 0)) for w in gw1]
        + [pl.BlockSpec((BN, 16), lambda i: (i, 0))] * 2,
        out_shape=[jax.ShapeDtypeStruct((N1, w), f32) for w in gw1]
        + [jax.ShapeDtypeStruct((N1, 16), f32)] * 2,
    )(xpad, Wg1, a1p, b1p)
    xg1, as1, ad1 = list(prep_out[:HEADS]), prep_out[HEADS], prep_out[HEADS + 1]

    outg1, den1 = _sc_edge(src, dst, as1, ad1, xg1, H1, L1_LANES)

    gw23 = (2 * H2,) * HEADS
    mid = pl.pallas_call(
        _tc_mid_body,
        grid=(NBLK,),
        in_specs=[pl.BlockSpec((2, BN, H1), lambda i: (0, i, 0))] * HEADS
        + [
            pl.BlockSpec((2, BN, 16), lambda i: (0, i, 0)),
            pl.BlockSpec((1, H1), lambda i: (0, 0)),
            pl.BlockSpec((H1, sum(gw23)), lambda i: (0, 0)),
            pl.BlockSpec((16, H2), lambda i: (0, 0)),
            pl.BlockSpec((16, H2), lambda i: (0, 0)),
        ],
        out_specs=[pl.BlockSpec((BN, w), lambda i: (i, 0)) for w in gw23]
        + [pl.BlockSpec((BN, 16), lambda i: (i, 0))] * 2,
        out_shape=[jax.ShapeDtypeStruct((N1, w), f32) for w in gw23]
        + [jax.ShapeDtypeStruct((N1, 16), f32)] * 2,
    )(*outg1, den1, b1[None, :], W23, a23p, b23p)
    xg23, as23, ad23 = list(mid[:HEADS]), mid[HEADS], mid[HEADS + 1]

    outg23, den23 = _sc_edge(src, dst, as23, ad23, xg23, H2, L23_LANES)

    mu_pad, lv_pad = pl.pallas_call(
        _tc_final_body,
        grid=(NBLK,),
        in_specs=[pl.BlockSpec((2, BN, 2 * H2), lambda i: (0, i, 0))] * HEADS
        + [
            pl.BlockSpec((2, BN, 16), lambda i: (0, i, 0)),
            pl.BlockSpec((1, H2), lambda i: (0, 0)),
            pl.BlockSpec((1, H2), lambda i: (0, 0)),
        ],
        out_specs=[pl.BlockSpec((BN, H2), lambda i: (i, 0))] * 2,
        out_shape=[jax.ShapeDtypeStruct((N1, H2), f32)] * 2,
    )(*outg23, den23, b2[None, :], b3[None, :])
    mu = mu_pad[:N]
    logvar = lv_pad[:N]
    return (mu, mu, logvar)


def _tc_prep_wrap(score_map, C, gw, x_ref, Wg_ref, apad_ref, bpad_ref,
                  *out_refs):
    _prep_block(x_ref[...], Wg_ref, apad_ref, bpad_ref, score_map, C,
                out_refs)


def _sc_edge(src, dst, as_tab, ad_tab, xg_list, C, group_lanes):
    ng = len(xg_list)
    WACC = xg_list[0].shape[-1]
    mesh = plsc.VectorSubcoreMesh(core_axis_name="c", subcore_axis_name="s")

    @functools.partial(
        pl.kernel,
        mesh=mesh,
        compiler_params=pltpu.CompilerParams(use_tc_tiling_on_sc=False),
        out_type=[jax.ShapeDtypeStruct((2, N1, WACC), jnp.float32)] * ng
        + [
            jax.ShapeDtypeStruct((2, N1, 16), jnp.float32),
            jax.ShapeDtypeStruct((NCHT, K, 16), jnp.float32),
        ],
        scratch_types=[
            pltpu.VMEM_SHARED((N1, WACC), jnp.float32),
            pltpu.VMEM_SHARED((N1, 16), jnp.float32),
            pltpu.VMEM((NCH, K), jnp.int32),
            pltpu.VMEM((NCH, K), jnp.int32),
            pltpu.VMEM((3, K, 16), jnp.float32),
            pltpu.VMEM((3, K, 16), jnp.float32),
            pltpu.VMEM((3, K, 16), jnp.float32),
            pltpu.VMEM((3, K, WACC), jnp.float32),
            pltpu.VMEM((ZR, WACC), jnp.float32),
            pltpu.VMEM((ZR, 16), jnp.float32),
            pltpu.VMEM((K,), jnp.int32),
            pltpu.VMEM((K,), jnp.int32),
            pltpu.VMEM((K,), jnp.int32),
        ] + [pltpu.SemaphoreType.DMA] * 12,
    )
    def k(src_hbm, dst_hbm, as_hbm, ad_hbm, *rest):
        xg_hbm = rest[:ng]
        outp_hbm = rest[ng:2 * ng]
        den_hbm, w_hbm = rest[2 * ng], rest[2 * ng + 1]
        (out_sp, den_sp, srcv, dstv, asr, adr, wvr, rowsv, zbo, zbd,
         dst0, dst1, dst2) = rest[2 * ng + 2:2 * ng + 15]
        sems = rest[2 * ng + 15:]
        dst_st = (dst0, dst1, dst2)
        sem_a = sems[0:3]
        sem_b = sems[3:6]
        sem_c = sems[6:9]
        sem_d = sems[9:12]
        cid = lax.axis_index("c")
        sid = lax.axis_index("s")
        wid = cid * 16 + sid
        cbase = wid * NCH
        lo = sid * RPT

        pltpu.sync_copy(src_hbm.at[pl.ds(cbase, NCH)], srcv)
        pltpu.sync_copy(dst_hbm.at[pl.ds(cbase, NCH)], dstv)

        zvec16 = jnp.zeros((16,), jnp.float32)

        def zfill(j, _):
            for cc in range(WACC // 16):
                zbo[j, pl.ds(cc * 16, 16)] = zvec16
            zbd[j, :] = zvec16
            return 0
        lax.fori_loop(0, ZR, zfill, 0, unroll=4)

        for t in range(RPT // ZR):
            pltpu.sync_copy(zbd, den_sp.at[pl.ds(lo + t * ZR, ZR)])
        plsc.subcore_barrier()

        def b_gather(i, b):
            pltpu.async_copy(as_hbm.at[srcv.at[i]], asr.at[b], sem_a[b])
            pltpu.async_copy(ad_hbm.at[dstv.at[i]], adr.at[b], sem_b[b])
            pltpu.async_copy(dst_hbm.at[cbase + i], dst_st[b], sem_d[b])

        b_gather(0, 0)
        b_gather(1, 1)

        def b_slot(t, b):
            i = 3 * t + b
            pltpu.make_async_copy(as_hbm.at[srcv.at[i]], asr.at[b],
                                  sem_a[b]).wait()
            pltpu.make_async_copy(ad_hbm.at[dstv.at[i]], adr.at[b],
                                  sem_b[b]).wait()

            def ej(j, _):
                e = asr[b, j, :] + adr[b, j, :]
                e = jnp.where(e >= 0.0, e, 0.2 * e)
                wvr[b, j, :] = jnp.exp(e)
                return 0
            lax.fori_loop(0, K, ej, 0, unroll=8)
            pltpu.async_copy(wvr.at[b], w_hbm.at[cbase + i], sem_c[b])
            pltpu.make_async_copy(dst_hbm.at[cbase + i], dst_st[b],
                                  sem_d[b]).wait()
            pltpu.sync_copy(wvr.at[b], den_sp.at[dst_st[b]], add=True)
            b2 = (b + 2) % 3

            def drain_w():
                pltpu.make_async_copy(wvr.at[b2], w_hbm.at[cbase + i - 1],
                                      sem_c[b2]).wait()

            if b == 0:
                @pl.when(t > 0)
                def _():
                    drain_w()
                b_gather(i + 2, b2)
            elif b == 2:
                drain_w()

                @pl.when(t < NT - 1)
                def _():
                    b_gather(i + 2, b2)
            else:
                drain_w()

                @pl.when(t < NT - 1)
                def _():
                    b_gather(i + 2, b2)

        def b_body(t, _):
            for b in range(3):
                b_slot(t, b)
            return 0
        lax.fori_loop(0, NT, b_body, 0)
        pltpu.make_async_copy(wvr.at[2], w_hbm.at[cbase + NCH - 1],
                              sem_c[2]).wait()
        plsc.subcore_barrier()
        pltpu.sync_copy(den_sp.at[pl.ds(lo, RPT)],
                        den_hbm.at[cid, pl.ds(lo, RPT)])

        for g in range(ng):
            lanes = group_lanes[g]
            for t in range(RPT // ZR):
                pltpu.sync_copy(zbo, out_sp.at[pl.ds(lo + t * ZR, ZR)])
            plsc.subcore_barrier()

            def c_gather(i, b):
                pltpu.async_copy(xg_hbm[g].at[srcv.at[i]], rowsv.at[b],
                                 sem_a[b])
                pltpu.async_copy(w_hbm.at[cbase + i], wvr.at[b], sem_b[b])
                pltpu.async_copy(dst_hbm.at[cbase + i], dst_st[b], sem_d[b])

            c_gather(0, 0)
            c_gather(1, 1)

            def c_slot(t, b):
                i = 3 * t + b
                pltpu.make_async_copy(xg_hbm[g].at[srcv.at[i]], rowsv.at[b],
                                      sem_a[b]).wait()
                pltpu.make_async_copy(w_hbm.at[cbase + i], wvr.at[b],
                                      sem_b[b]).wait()

                def ej(j, _):
                    wv = wvr[b, j, :]
                    for bi, lane in enumerate(lanes):
                        wb = wv[lane]
                        for cc in range(C // 16):
                            sl = pl.ds(bi * C + cc * 16, 16)
                            rowsv[b, j, sl] = rowsv[b, j, sl] * wb
                    return 0
                lax.fori_loop(0, K, ej, 0, unroll=8)
                pltpu.make_async_copy(dst_hbm.at[cbase + i], dst_st[b],
                                      sem_d[b]).wait()
                pltpu.async_copy(rowsv.at[b], out_sp.at[dst_st[b]],
                                 sem_c[b], add=True)
                b2 = (b + 2) % 3

                def drain_s():
                    pltpu.make_async_copy(rowsv.at[b2],
                                          out_sp.at[dst_st[b2]],
                                          sem_c[b2]).wait()

                if b == 0:
                    @pl.when(t > 0)
                    def _():
                        drain_s()
                    c_gather(i + 2, b2)
                else:
                    drain_s()

                    @pl.when(t < NT - 1)
                    def _():
                        c_gather(i + 2, b2)

            def c_body(t, _):
                for b in range(3):
                    c_slot(t, b)
                return 0
            lax.fori_loop(0, NT, c_body, 0)
            pltpu.make_async_copy(rowsv.at[2], out_sp.at[dst_st[2]],
                                  sem_c[2]).wait()
            plsc.subcore_barrier()
            pltpu.sync_copy(out_sp.at[pl.ds(lo, RPT)],
                            outp_hbm[g].at[cid, pl.ds(lo, RPT)])

    outs = k(src, dst, as_tab, ad_tab, *xg_list)
    return list(outs[:ng]), outs[ng]


# speedup vs baseline: 67.1252x; 1.0105x over previous
import functools

import jax
import jax.numpy as jnp
from jax import lax
from jax.experimental import pallas as pl
from jax.experimental.pallas import tpu as pltpu
from jax.experimental.pallas import tpu_sc as plsc

N = 10000
E = 320000
D_IN = 128
H1 = 64
H2 = 32
HEADS = 5

N1 = 10112
BN = 632
NBLK = N1 // BN
K = 128
NW = 32
E1 = E + N
EPAD = ((E1 + NW * K - 1) // (NW * K)) * (NW * K)
NCHT = EPAD // K
NCH = NCHT // NW
NT = NCH // 3
RPT = N1 // 16
ZR = 158
EPS = 1e-16

L1_LANES = ((0,), (1,), (2,), (3,), (4,))
L23_LANES = tuple((h, 8 + h) for h in range(HEADS))


def _prep_block(xb, Wg_ref, apad_ref, bpad_ref, score_map, C, out_refs):
    ngroups = len(out_refs) - 2
    as_ref, ad_ref = out_refs[ngroups], out_refs[ngroups + 1]
    xg = []
    off = 0
    for g in range(ngroups):
        wg = out_refs[g].shape[-1]
        x_out = jnp.dot(xb, Wg_ref[:, off:off + wg],
                        preferred_element_type=jnp.float32)
        out_refs[g][...] = x_out
        xg.append(x_out)
        off += wg
    dn = (((1,), (1,)), ((), ()))
    zcol = jnp.zeros((xb.shape[0], 1), jnp.float32)
    acols, bcols = [zcol] * 16, [zcol] * 16
    for lane, g, coff in score_map:
        blk = xg[g][:, coff:coff + C]
        acols[lane] = lax.dot_general(blk, apad_ref[lane:lane + 1, :], dn,
                                      preferred_element_type=jnp.float32)
        bcols[lane] = lax.dot_general(blk, bpad_ref[lane:lane + 1, :], dn,
                                      preferred_element_type=jnp.float32)
    as_ref[...] = jnp.concatenate(acols, axis=1)
    ad_ref[...] = jnp.concatenate(bcols, axis=1)


def _combine(groups, lanes, cw, den, eps_lanes):
    dsum = den[0] + den[1]
    acc = None
    for (g, bi, lane) in eps_lanes:
        oh = groups[g][0][:, bi * cw:(bi + 1) * cw] \
            + groups[g][1][:, bi * cw:(bi + 1) * cw]
        dh = dsum[:, lane:lane + 1] + EPS
        term = oh / dh
        acc = term if acc is None else acc + term
    return acc * (1.0 / HEADS)


def _tc_mid_body(og0, og1, og2, og3, og4, den_ref, b1_ref, W23_ref,
                 apad_ref, bpad_ref, *out_refs):
    groups = [(og[0], og[1]) for og in (og0, og1, og2, og3, og4)]
    triples = [(h, 0, h) for h in range(HEADS)]
    h1 = _combine(groups, L1_LANES, H1, den_ref[...], triples) + b1_ref[...]
    h1 = jnp.maximum(h1, 0.0)
    score_map = tuple([(h, h, 0) for h in range(HEADS)]
                      + [(8 + h, h, H2) for h in range(HEADS)])
    _prep_block(h1, W23_ref, apad_ref, bpad_ref, score_map, H2, out_refs)


def _tc_final_body(g0, g1, g2, g3, g4, den_ref, b2_ref, b3_ref,
                   mu_ref, lv_ref):
    groups = [(g0[0], g0[1]), (g1[0], g1[1]), (g2[0], g2[1]),
              (g3[0], g3[1]), (g4[0], g4[1])]
    den = den_ref[...]
    mu_t = [(h, 0, h) for h in range(HEADS)]
    lv_t = [(h, 1, 8 + h) for h in range(HEADS)]
    mu_ref[...] = _combine(groups, L23_LANES, H2, den, mu_t) + b2_ref[...]
    lv_ref[...] = _combine(groups, L23_LANES, H2, den, lv_t) + b3_ref[...]


def kernel(x, adj, W1, a_src1, a_dst1, b1, W2, a_src2, a_dst2, b2,
           W3, a_src3, a_dst3, b3):
    f32 = jnp.float32
    adj32 = adj.astype(jnp.int32)
    loops = jnp.arange(N, dtype=jnp.int32)
    padv = jnp.full((EPAD - E1,), N, jnp.int32)
    src = jnp.concatenate([adj32[0], loops, padv]).reshape(NCHT, K)
    dst = jnp.concatenate([adj32[1], loops, padv]).reshape(NCHT, K)
    xpad = jnp.pad(x, ((0, N1 - N), (0, 0)))
    Wg1 = W1
    a1p = jnp.zeros((16, H1), f32).at[:HEADS].set(a_src1)
    b1p = jnp.zeros((16, H1), f32).at[:HEADS].set(a_dst1)
    W23 = jnp.concatenate([W2.reshape(H1, HEADS, H2),
                           W3.reshape(H1, HEADS, H2)], axis=2)
    W23 = W23.reshape(H1, 2 * HEADS * H2)
    a23p = jnp.zeros((16, H2), f32).at[:HEADS].set(a_src2).at[8:8 + HEADS].set(a_src3)
    b23p = jnp.zeros((16, H2), f32).at[:HEADS].set(a_dst2).at[8:8 + HEADS].set(a_dst3)

    score_map1 = tuple((h, h, 0) for h in range(HEADS))
    gw1 = (H1,) * HEADS
    prep_out = pl.pallas_call(
        functools.partial(_tc_prep_wrap, score_map1, H1, gw1),
        grid=(NBLK,),
        in_specs=[
            pl.BlockSpec((BN, D_IN), lambda i: (i, 0)),
            pl.BlockSpec((D_IN, sum(gw1)), lambda i: (0, 0)),
            pl.BlockSpec((16, H1), lambda i: (0, 0)),
            pl.BlockSpec((16, H1), lambda i: (0, 0)),
        ],
        out_specs=[pl.BlockSpec((BN, w), lambda i: (i, 0)) for w in gw1]
        + [pl.BlockSpec((BN, 16), lambda i: (i, 0))] * 2,
        out_shape=[jax.ShapeDtypeStruct((N1, w), f32) for w in gw1]
        + [jax.ShapeDtypeStruct((N1, 16), f32)] * 2,
    )(xpad, Wg1, a1p, b1p)
    xg1, as1, ad1 = (list(prep_out[:HEADS]), prep_out[HEADS],
                     prep_out[HEADS + 1])

    outg1, den1 = _sc_edge(src, dst, as1, ad1, xg1, H1, L1_LANES)

    gw23 = (2 * H2,) * HEADS
    mid = pl.pallas_call(
        _tc_mid_body,
        grid=(NBLK,),
        in_specs=[pl.BlockSpec((2, BN, H1), lambda i: (0, i, 0))] * HEADS
        + [
            pl.BlockSpec((2, BN, 16), lambda i: (0, i, 0)),
            pl.BlockSpec((1, H1), lambda i: (0, 0)),
            pl.BlockSpec((H1, sum(gw23)), lambda i: (0, 0)),
            pl.BlockSpec((16, H2), lambda i: (0, 0)),
            pl.BlockSpec((16, H2), lambda i: (0, 0)),
        ],
        out_specs=[pl.BlockSpec((BN, w), lambda i: (i, 0)) for w in gw23]
        + [pl.BlockSpec((BN, 16), lambda i: (i, 0))] * 2,
        out_shape=[jax.ShapeDtypeStruct((N1, w), f32) for w in gw23]
        + [jax.ShapeDtypeStruct((N1, 16), f32)] * 2,
    )(*outg1, den1, b1[None, :], W23, a23p, b23p)
    xg23, as23, ad23 = list(mid[:HEADS]), mid[HEADS], mid[HEADS + 1]

    outg23, den23 = _sc_edge(src, dst, as23, ad23, xg23, H2, L23_LANES)

    mu_pad, lv_pad = pl.pallas_call(
        _tc_final_body,
        grid=(NBLK,),
        in_specs=[pl.BlockSpec((2, BN, 2 * H2), lambda i: (0, i, 0))] * HEADS
        + [
            pl.BlockSpec((2, BN, 16), lambda i: (0, i, 0)),
            pl.BlockSpec((1, H2), lambda i: (0, 0)),
            pl.BlockSpec((1, H2), lambda i: (0, 0)),
        ],
        out_specs=[pl.BlockSpec((BN, H2), lambda i: (i, 0))] * 2,
        out_shape=[jax.ShapeDtypeStruct((N1, H2), f32)] * 2,
    )(*outg23, den23, b2[None, :], b3[None, :])
    mu = mu_pad[:N]
    logvar = lv_pad[:N]
    return (mu, mu, logvar)


def _tc_prep_wrap(score_map, C, gw, x_ref, Wg_ref, apad_ref, bpad_ref,
                  *out_refs):
    _prep_block(x_ref[...], Wg_ref, apad_ref, bpad_ref, score_map, C,
                out_refs)


def _sc_edge(src, dst, as_tab, ad_tab, xg_list, C, group_lanes):
    ng = len(xg_list)
    WACC = xg_list[0].shape[-1]
    mesh = plsc.VectorSubcoreMesh(core_axis_name="c", subcore_axis_name="s")

    @functools.partial(
        pl.kernel,
        mesh=mesh,
        compiler_params=pltpu.CompilerParams(use_tc_tiling_on_sc=False),
        out_type=[jax.ShapeDtypeStruct((2, N1, WACC), jnp.float32)] * ng
        + [
            jax.ShapeDtypeStruct((2, N1, 16), jnp.float32),
            jax.ShapeDtypeStruct((NCHT, K, 16), jnp.float32),
        ],
        scratch_types=[
            pltpu.VMEM_SHARED((N1, WACC), jnp.float32),
            pltpu.VMEM_SHARED((N1, 16), jnp.float32),
            pltpu.VMEM((NCH, K), jnp.int32),
            pltpu.VMEM((NCH, K), jnp.int32),
            pltpu.VMEM((3, K, 16), jnp.float32),
            pltpu.VMEM((3, K, 16), jnp.float32),
            pltpu.VMEM((3, K, 16), jnp.float32),
            pltpu.VMEM((3, K, WACC), jnp.float32),
            pltpu.VMEM((ZR, WACC), jnp.float32),
            pltpu.VMEM((ZR, 16), jnp.float32),
            pltpu.VMEM((K,), jnp.int32),
            pltpu.VMEM((K,), jnp.int32),
            pltpu.VMEM((K,), jnp.int32),
        ] + [pltpu.SemaphoreType.DMA] * 12,
    )
    def k(src_hbm, dst_hbm, as_hbm, ad_hbm, *rest):
        xg_hbm = rest[:ng]
        outp_hbm = rest[ng:2 * ng]
        den_hbm, w_hbm = rest[2 * ng], rest[2 * ng + 1]
        (out_sp, den_sp, srcv, dstv, asr, adr, wvr, rowsv, zbo, zbd,
         dst0, dst1, dst2) = rest[2 * ng + 2:2 * ng + 15]
        sems = rest[2 * ng + 15:]
        dst_st = (dst0, dst1, dst2)
        sem_a = sems[0:3]
        sem_b = sems[3:6]
        sem_c = sems[6:9]
        sem_d = sems[9:12]
        cid = lax.axis_index("c")
        sid = lax.axis_index("s")
        wid = cid * 16 + sid
        cbase = wid * NCH
        lo = sid * RPT

        pltpu.sync_copy(src_hbm.at[pl.ds(cbase, NCH)], srcv)
        pltpu.sync_copy(dst_hbm.at[pl.ds(cbase, NCH)], dstv)

        zvec16 = jnp.zeros((16,), jnp.float32)

        def zfill(j, _):
            for cc in range(WACC // 16):
                zbo[j, pl.ds(cc * 16, 16)] = zvec16
            zbd[j, :] = zvec16
            return 0
        lax.fori_loop(0, ZR, zfill, 0, unroll=4)

        for t in range(RPT // ZR):
            pltpu.sync_copy(zbd, den_sp.at[pl.ds(lo + t * ZR, ZR)])
        plsc.subcore_barrier()

        def b_gather(i, b):
            pltpu.async_copy(as_hbm.at[srcv.at[i]], asr.at[b], sem_a[b])
            pltpu.async_copy(ad_hbm.at[dstv.at[i]], adr.at[b], sem_b[b])
            pltpu.async_copy(dst_hbm.at[cbase + i], dst_st[b], sem_d[b])

        b_gather(0, 0)
        b_gather(1, 1)

        def b_slot(t, b):
            i = 3 * t + b
            pltpu.make_async_copy(as_hbm.at[srcv.at[i]], asr.at[b],
                                  sem_a[b]).wait()
            pltpu.make_async_copy(ad_hbm.at[dstv.at[i]], adr.at[b],
                                  sem_b[b]).wait()

            def ej(j, _):
                e = asr[b, j, :] + adr[b, j, :]
                e = jnp.where(e >= 0.0, e, 0.2 * e)
                wvr[b, j, :] = jnp.exp(e)
                return 0
            lax.fori_loop(0, K, ej, 0, unroll=8)
            pltpu.async_copy(wvr.at[b], w_hbm.at[cbase + i], sem_c[b])
            pltpu.make_async_copy(dst_hbm.at[cbase + i], dst_st[b],
                                  sem_d[b]).wait()
            pltpu.sync_copy(wvr.at[b], den_sp.at[dst_st[b]], add=True)
            b2 = (b + 2) % 3

            def drain_w():
                pltpu.make_async_copy(wvr.at[b2], w_hbm.at[cbase + i - 1],
                                      sem_c[b2]).wait()

            if b == 0:
                @pl.when(t > 0)
                def _():
                    drain_w()
                b_gather(i + 2, b2)
            elif b == 2:
                drain_w()

                @pl.when(t < NT - 1)
                def _():
                    b_gather(i + 2, b2)
            else:
                drain_w()

                @pl.when(t < NT - 1)
                def _():
                    b_gather(i + 2, b2)

        def b_body(t, _):
            for b in range(3):
                b_slot(t, b)
            return 0
        lax.fori_loop(0, NT, b_body, 0)
        pltpu.make_async_copy(wvr.at[2], w_hbm.at[cbase + NCH - 1],
                              sem_c[2]).wait()
        plsc.subcore_barrier()
        pltpu.sync_copy(den_sp.at[pl.ds(lo, RPT)],
                        den_hbm.at[cid, pl.ds(lo, RPT)])

        for g in range(ng):
            lanes = group_lanes[g]
            for t in range(RPT // ZR):
                pltpu.sync_copy(zbo, out_sp.at[pl.ds(lo + t * ZR, ZR)])
            plsc.subcore_barrier()

            def c_gather(i, b):
                pltpu.async_copy(xg_hbm[g].at[srcv.at[i]], rowsv.at[b],
                                 sem_a[b])
                pltpu.async_copy(w_hbm.at[cbase + i], wvr.at[b], sem_b[b])
                pltpu.async_copy(dst_hbm.at[cbase + i], dst_st[b], sem_d[b])

            c_gather(0, 0)
            c_gather(1, 1)

            def c_slot(t, b):
                i = 3 * t + b
                pltpu.make_async_copy(xg_hbm[g].at[srcv.at[i]], rowsv.at[b],
                                      sem_a[b]).wait()
                pltpu.make_async_copy(w_hbm.at[cbase + i], wvr.at[b],
                                      sem_b[b]).wait()

                def ej(j, _):
                    wv = wvr[b, j, :]
                    for bi, lane in enumerate(lanes):
                        wb = wv[lane]
                        for cc in range(C // 16):
                            sl = pl.ds(bi * C + cc * 16, 16)
                            rowsv[b, j, sl] = rowsv[b, j, sl] * wb
                    return 0
                lax.fori_loop(0, K, ej, 0, unroll=8)
                pltpu.make_async_copy(dst_hbm.at[cbase + i], dst_st[b],
                                      sem_d[b]).wait()
                pltpu.async_copy(rowsv.at[b], out_sp.at[dst_st[b]],
                                 sem_c[b], add=True)
                b2 = (b + 2) % 3

                def drain_s():
                    pltpu.make_async_copy(rowsv.at[b2],
                                          out_sp.at[dst_st[b2]],
                                          sem_c[b2]).wait()

                if b == 0:
                    @pl.when(t > 0)
                    def _():
                        drain_s()
                    c_gather(i + 2, b2)
                else:
                    drain_s()

                    @pl.when(t < NT - 1)
                    def _():
                        c_gather(i + 2, b2)

            def c_body(t, _):
                for b in range(3):
                    c_slot(t, b)
                return 0
            lax.fori_loop(0, NT, c_body, 0)
            pltpu.make_async_copy(rowsv.at[2], out_sp.at[dst_st[2]],
                                  sem_c[2]).wait()
            plsc.subcore_barrier()
            pltpu.sync_copy(out_sp.at[pl.ds(lo, RPT)],
                            outp_hbm[g].at[cid, pl.ds(lo, RPT)])

    outs = k(src, dst, as_tab, ad_tab, *xg_list)
    return list(outs[:ng]), outs[ng]


# speedup vs baseline: 70.2355x; 1.0463x over previous
import functools

import jax
import jax.numpy as jnp
from jax import lax
from jax.experimental import pallas as pl
from jax.experimental.pallas import tpu as pltpu
from jax.experimental.pallas import tpu_sc as plsc

N = 10000
E = 320000
D_IN = 128
H1 = 64
H2 = 32
HEADS = 5

N1 = 10112
BN = 632
NBLK = N1 // BN
K = 128
NW = 32
E1 = E + N
EPAD = ((E1 + NW * K - 1) // (NW * K)) * (NW * K)
NCHT = EPAD // K
NCH = NCHT // NW
NT = NCH // 3
RPT = N1 // 16
ZR = 158
EPS = 1e-16

L1_LANES = ((0,), (1,), (2,), (3,), (4,))
L23_LANES = tuple((h, 8 + h) for h in range(HEADS))


def _prep_block(xb, Wg_ref, apad_ref, bpad_ref, score_map, C, out_refs):
    ngroups = len(out_refs) - 2
    as_ref, ad_ref = out_refs[ngroups], out_refs[ngroups + 1]
    xg = []
    off = 0
    for g in range(ngroups):
        wg = out_refs[g].shape[-1]
        x_out = jnp.dot(xb, Wg_ref[:, off:off + wg],
                        preferred_element_type=jnp.float32)
        out_refs[g][...] = x_out
        xg.append(x_out)
        off += wg
    dn = (((1,), (1,)), ((), ()))
    zcol = jnp.zeros((xb.shape[0], 1), jnp.float32)
    acols, bcols = [zcol] * 16, [zcol] * 16
    for lane, g, coff in score_map:
        blk = xg[g][:, coff:coff + C]
        acols[lane] = lax.dot_general(blk, apad_ref[lane:lane + 1, :], dn,
                                      preferred_element_type=jnp.float32)
        bcols[lane] = lax.dot_general(blk, bpad_ref[lane:lane + 1, :], dn,
                                      preferred_element_type=jnp.float32)
    as_ref[...] = jnp.concatenate(acols, axis=1)
    ad_ref[...] = jnp.concatenate(bcols, axis=1)


def _combine(groups, lanes, cw, den, eps_lanes):
    dsum = den[0] + den[1]
    acc = None
    for (g, bi, lane) in eps_lanes:
        oh = groups[g][0][:, bi * cw:(bi + 1) * cw] \
            + groups[g][1][:, bi * cw:(bi + 1) * cw]
        dh = dsum[:, lane:lane + 1] + EPS
        term = oh / dh
        acc = term if acc is None else acc + term
    return acc * (1.0 / HEADS)


def _tc_mid_body(og0, og1, og2, og3, og4, den_ref, b1_ref, W23_ref,
                 apad_ref, bpad_ref, *out_refs):
    groups = [(og[0], og[1]) for og in (og0, og1, og2, og3, og4)]
    triples = [(h, 0, h) for h in range(HEADS)]
    h1 = _combine(groups, L1_LANES, H1, den_ref[...], triples) + b1_ref[...]
    h1 = jnp.maximum(h1, 0.0)
    score_map = tuple([(h, h, 0) for h in range(HEADS)]
                      + [(8 + h, h, H2) for h in range(HEADS)])
    _prep_block(h1, W23_ref, apad_ref, bpad_ref, score_map, H2, out_refs)


def _tc_final_body(g0, g1, g2, g3, g4, den_ref, b2_ref, b3_ref,
                   mu_ref, lv_ref):
    groups = [(g0[0], g0[1]), (g1[0], g1[1]), (g2[0], g2[1]),
              (g3[0], g3[1]), (g4[0], g4[1])]
    den = den_ref[...]
    mu_t = [(h, 0, h) for h in range(HEADS)]
    lv_t = [(h, 1, 8 + h) for h in range(HEADS)]
    mu_ref[...] = _combine(groups, L23_LANES, H2, den, mu_t) + b2_ref[...]
    lv_ref[...] = _combine(groups, L23_LANES, H2, den, lv_t) + b3_ref[...]


def kernel(x, adj, W1, a_src1, a_dst1, b1, W2, a_src2, a_dst2, b2,
           W3, a_src3, a_dst3, b3):
    f32 = jnp.float32
    adj32 = adj.astype(jnp.int32)
    loops = jnp.arange(N, dtype=jnp.int32)
    padv = jnp.full((EPAD - E1,), N, jnp.int32)
    src = jnp.concatenate([adj32[0], loops, padv]).reshape(NCHT, K)
    dst = jnp.concatenate([adj32[1], loops, padv]).reshape(NCHT, K)
    xpad = jnp.pad(x, ((0, N1 - N), (0, 0)))
    Wg1 = W1
    a1p = jnp.zeros((16, H1), f32).at[:HEADS].set(a_src1)
    b1p = jnp.zeros((16, H1), f32).at[:HEADS].set(a_dst1)
    W23 = jnp.concatenate([W2.reshape(H1, HEADS, H2),
                           W3.reshape(H1, HEADS, H2)], axis=2)
    W23 = W23.reshape(H1, 2 * HEADS * H2)
    a23p = jnp.zeros((16, H2), f32).at[:HEADS].set(a_src2).at[8:8 + HEADS].set(a_src3)
    b23p = jnp.zeros((16, H2), f32).at[:HEADS].set(a_dst2).at[8:8 + HEADS].set(a_dst3)

    score_map1 = tuple((h, h, 0) for h in range(HEADS))
    gw1 = (H1,) * HEADS
    prep_out = pl.pallas_call(
        functools.partial(_tc_prep_wrap, score_map1, H1, gw1),
        grid=(NBLK,),
        in_specs=[
            pl.BlockSpec((BN, D_IN), lambda i: (i, 0)),
            pl.BlockSpec((D_IN, sum(gw1)), lambda i: (0, 0)),
            pl.BlockSpec((16, H1), lambda i: (0, 0)),
            pl.BlockSpec((16, H1), lambda i: (0, 0)),
        ],
        out_specs=[pl.BlockSpec((BN, w), lambda i: (i, 0)) for w in gw1]
        + [pl.BlockSpec((BN, 16), lambda i: (i, 0))] * 2,
        out_shape=[jax.ShapeDtypeStruct((N1, w), f32) for w in gw1]
        + [jax.ShapeDtypeStruct((N1, 16), f32)] * 2,
    )(xpad, Wg1, a1p, b1p)
    xg1, as1, ad1 = (list(prep_out[:HEADS]), prep_out[HEADS],
                     prep_out[HEADS + 1])

    outg1, den1 = _sc_edge(src, dst, as1, ad1, xg1, H1, L1_LANES)

    gw23 = (2 * H2,) * HEADS
    mid = pl.pallas_call(
        _tc_mid_body,
        grid=(NBLK,),
        in_specs=[pl.BlockSpec((2, BN, H1), lambda i: (0, i, 0))] * HEADS
        + [
            pl.BlockSpec((2, BN, 16), lambda i: (0, i, 0)),
            pl.BlockSpec((1, H1), lambda i: (0, 0)),
            pl.BlockSpec((H1, sum(gw23)), lambda i: (0, 0)),
            pl.BlockSpec((16, H2), lambda i: (0, 0)),
            pl.BlockSpec((16, H2), lambda i: (0, 0)),
        ],
        out_specs=[pl.BlockSpec((BN, w), lambda i: (i, 0)) for w in gw23]
        + [pl.BlockSpec((BN, 16), lambda i: (i, 0))] * 2,
        out_shape=[jax.ShapeDtypeStruct((N1, w), f32) for w in gw23]
        + [jax.ShapeDtypeStruct((N1, 16), f32)] * 2,
    )(*outg1, den1, b1[None, :], W23, a23p, b23p)
    xg23, as23, ad23 = list(mid[:HEADS]), mid[HEADS], mid[HEADS + 1]

    outg23, den23 = _sc_edge(src, dst, as23, ad23, xg23, H2, L23_LANES)

    mu_pad, lv_pad = pl.pallas_call(
        _tc_final_body,
        grid=(NBLK,),
        in_specs=[pl.BlockSpec((2, BN, 2 * H2), lambda i: (0, i, 0))] * HEADS
        + [
            pl.BlockSpec((2, BN, 16), lambda i: (0, i, 0)),
            pl.BlockSpec((1, H2), lambda i: (0, 0)),
            pl.BlockSpec((1, H2), lambda i: (0, 0)),
        ],
        out_specs=[pl.BlockSpec((BN, H2), lambda i: (i, 0))] * 2,
        out_shape=[jax.ShapeDtypeStruct((N1, H2), f32)] * 2,
    )(*outg23, den23, b2[None, :], b3[None, :])
    mu = mu_pad[:N]
    logvar = lv_pad[:N]
    return (mu, mu, logvar)


def _tc_prep_wrap(score_map, C, gw, x_ref, Wg_ref, apad_ref, bpad_ref,
                  *out_refs):
    _prep_block(x_ref[...], Wg_ref, apad_ref, bpad_ref, score_map, C,
                out_refs)


def _sc_edge(src, dst, as_tab, ad_tab, xg_list, C, group_lanes):
    ng = len(xg_list)
    WACC = xg_list[0].shape[-1]
    mesh = plsc.VectorSubcoreMesh(core_axis_name="c", subcore_axis_name="s")

    @functools.partial(
        pl.kernel,
        mesh=mesh,
        compiler_params=pltpu.CompilerParams(use_tc_tiling_on_sc=False),
        out_type=[jax.ShapeDtypeStruct((2, N1, WACC), jnp.float32)] * ng
        + [
            jax.ShapeDtypeStruct((2, N1, 16), jnp.float32),
            jax.ShapeDtypeStruct((NCHT, K, 16), jnp.float32),
        ],
        scratch_types=[
            pltpu.VMEM_SHARED((N1, WACC), jnp.float32),
            pltpu.VMEM_SHARED((N1, 16), jnp.float32),
            pltpu.VMEM((NCH, K), jnp.int32),
            pltpu.VMEM((NCH, K), jnp.int32),
            pltpu.VMEM((3, K, 16), jnp.float32),
            pltpu.VMEM((3, K, 16), jnp.float32),
            pltpu.VMEM((3, K, 16), jnp.float32),
            pltpu.VMEM((3, K, WACC), jnp.float32),
            pltpu.VMEM((ZR, WACC), jnp.float32),
            pltpu.VMEM((ZR, 16), jnp.float32),
            pltpu.VMEM((K,), jnp.int32),
            pltpu.VMEM((K,), jnp.int32),
            pltpu.VMEM((K,), jnp.int32),
        ] + [pltpu.SemaphoreType.DMA] * 12,
    )
    def k(src_hbm, dst_hbm, as_hbm, ad_hbm, *rest):
        xg_hbm = rest[:ng]
        outp_hbm = rest[ng:2 * ng]
        den_hbm, w_hbm = rest[2 * ng], rest[2 * ng + 1]
        (out_sp, den_sp, srcv, dstv, asr, adr, wvr, rowsv, zbo, zbd,
         dst0, dst1, dst2) = rest[2 * ng + 2:2 * ng + 15]
        sems = rest[2 * ng + 15:]
        dst_st = (dst0, dst1, dst2)
        sem_a = sems[0:3]
        sem_b = sems[3:6]
        sem_c = sems[6:9]
        sem_d = sems[9:12]
        cid = lax.axis_index("c")
        sid = lax.axis_index("s")
        wid = cid * 16 + sid
        cbase = wid * NCH
        lo = sid * RPT

        pltpu.sync_copy(src_hbm.at[pl.ds(cbase, NCH)], srcv)
        pltpu.sync_copy(dst_hbm.at[pl.ds(cbase, NCH)], dstv)

        zvec16 = jnp.zeros((16,), jnp.float32)

        def zfill(j, _):
            for cc in range(WACC // 16):
                zbo[j, pl.ds(cc * 16, 16)] = zvec16
            zbd[j, :] = zvec16
            return 0
        lax.fori_loop(0, ZR, zfill, 0, unroll=4)

        for t in range(RPT // ZR):
            pltpu.sync_copy(zbd, den_sp.at[pl.ds(lo + t * ZR, ZR)])
        plsc.subcore_barrier()

        def b_gather(i, b):
            pltpu.async_copy(as_hbm.at[srcv.at[i]], asr.at[b], sem_a[b])
            pltpu.async_copy(ad_hbm.at[dstv.at[i]], adr.at[b], sem_b[b])
            pltpu.async_copy(dst_hbm.at[cbase + i], dst_st[b], sem_d[b])

        b_gather(0, 0)
        b_gather(1, 1)

        def b_slot(t, b):
            i = 3 * t + b
            pltpu.make_async_copy(as_hbm.at[srcv.at[i]], asr.at[b],
                                  sem_a[b]).wait()
            pltpu.make_async_copy(ad_hbm.at[dstv.at[i]], adr.at[b],
                                  sem_b[b]).wait()

            def ej(j, _):
                e = asr[b, j, :] + adr[b, j, :]
                e = jnp.where(e >= 0.0, e, 0.2 * e)
                wvr[b, j, :] = jnp.exp(e)
                return 0
            lax.fori_loop(0, K, ej, 0, unroll=8)
            pltpu.async_copy(wvr.at[b], w_hbm.at[cbase + i], sem_c[b])
            pltpu.make_async_copy(dst_hbm.at[cbase + i], dst_st[b],
                                  sem_d[b]).wait()
            pltpu.sync_copy(wvr.at[b], den_sp.at[dst_st[b]], add=True)
            b2 = (b + 2) % 3

            def drain_w():
                pltpu.make_async_copy(wvr.at[b2], w_hbm.at[cbase + i - 1],
                                      sem_c[b2]).wait()

            if b == 0:
                @pl.when(t > 0)
                def _():
                    drain_w()
                b_gather(i + 2, b2)
            elif b == 2:
                drain_w()

                @pl.when(t < NT - 1)
                def _():
                    b_gather(i + 2, b2)
            else:
                drain_w()

                @pl.when(t < NT - 1)
                def _():
                    b_gather(i + 2, b2)

        def b_body(t, _):
            for b in range(3):
                b_slot(t, b)
            return 0
        lax.fori_loop(0, NT, b_body, 0)
        pltpu.make_async_copy(wvr.at[2], w_hbm.at[cbase + NCH - 1],
                              sem_c[2]).wait()
        plsc.subcore_barrier()
        pltpu.sync_copy(den_sp.at[pl.ds(lo, RPT)],
                        den_hbm.at[cid, pl.ds(lo, RPT)])

        for g in range(ng):
            lanes = group_lanes[g]
            for t in range(RPT // ZR):
                pltpu.sync_copy(zbo, out_sp.at[pl.ds(lo + t * ZR, ZR)])
            plsc.subcore_barrier()

            def c_gather(i, b):
                pltpu.async_copy(xg_hbm[g].at[srcv.at[i]], rowsv.at[b],
                                 sem_a[b])
                pltpu.async_copy(w_hbm.at[cbase + i], wvr.at[b], sem_b[b])
                pltpu.async_copy(dst_hbm.at[cbase + i], dst_st[b], sem_d[b])

            c_gather(0, 0)
            c_gather(1, 1)

            def c_slot(t, b):
                i = 3 * t + b
                pltpu.make_async_copy(xg_hbm[g].at[srcv.at[i]], rowsv.at[b],
                                      sem_a[b]).wait()
                pltpu.make_async_copy(w_hbm.at[cbase + i], wvr.at[b],
                                      sem_b[b]).wait()

                nsl = len(lanes) * (C // 16)

                def ej(u, _):
                    js = (2 * u, 2 * u + 1)
                    wrows = [wvr[b, j, :] for j in js]
                    vals = [[rowsv[b, j, pl.ds(s * 16, 16)]
                             for s in range(nsl)] for j in js]
                    outs = []
                    for p in range(2):
                        wbs = [wrows[p][lane] for lane in lanes]
                        outs.append([vals[p][s] * wbs[s // (C // 16)]
                                     for s in range(nsl)])
                    for p, j in enumerate(js):
                        for s in range(nsl):
                            rowsv[b, j, pl.ds(s * 16, 16)] = outs[p][s]
                    return 0
                lax.fori_loop(0, K // 2, ej, 0, unroll=4)
                pltpu.make_async_copy(dst_hbm.at[cbase + i], dst_st[b],
                                      sem_d[b]).wait()
                pltpu.async_copy(rowsv.at[b], out_sp.at[dst_st[b]],
                                 sem_c[b], add=True)
                b2 = (b + 2) % 3

                def drain_s():
                    pltpu.make_async_copy(rowsv.at[b2],
                                          out_sp.at[dst_st[b2]],
                                          sem_c[b2]).wait()

                if b == 0:
                    @pl.when(t > 0)
                    def _():
                        drain_s()
                    c_gather(i + 2, b2)
                else:
                    drain_s()

                    @pl.when(t < NT - 1)
                    def _():
                        c_gather(i + 2, b2)

            def c_body(t, _):
                for b in range(3):
                    c_slot(t, b)
                return 0
            lax.fori_loop(0, NT, c_body, 0)
            pltpu.make_async_copy(rowsv.at[2], out_sp.at[dst_st[2]],
                                  sem_c[2]).wait()
            plsc.subcore_barrier()
            pltpu.sync_copy(out_sp.at[pl.ds(lo, RPT)],
                            outp_hbm[g].at[cid, pl.ds(lo, RPT)])

    outs = k(src, dst, as_tab, ad_tab, *xg_list)
    return list(outs[:ng]), outs[ng]
